# baseline (speedup 1.0000x reference)
import functools

import jax
import jax.numpy as jnp
from jax import lax
from jax.experimental import pallas as pl
from jax.experimental.pallas import tpu as pltpu
from jax.experimental.pallas import tpu_sc as plsc

N = 10000
E = 320000
D = 128
H = 256
HH = 128
G = 64

NTILES = 16
WIN = 128
NW = 160
CH = 16
NCHUNK = NW // CH
EP = NTILES * NW * WIN
PAD = EP - E
NDUM = 240
ACC_ROWS = N + NDUM
ROWS_PER_TILE = 624
TAIL = N - NTILES * ROWS_PER_TILE
TAIL_OFF = NTILES * ROWS_PER_TILE
DEG_PER_TILE = ACC_ROWS // NTILES

BLK = 1000
NBLK = N // BLK


def _sc_mesh():
    return plsc.VectorSubcoreMesh(
        core_axis_name="c", subcore_axis_name="s", num_cores=2,
        num_subcores=NTILES)


def _deg_call(dst2d):
    @functools.partial(
        pl.kernel,
        out_type=jax.ShapeDtypeStruct((ACC_ROWS,), jnp.float32),
        mesh=_sc_mesh(),
        scratch_types=[
            pltpu.VMEM((NW, WIN), jnp.int32),
            pltpu.VMEM((WIN,), jnp.float32),
            pltpu.VMEM((DEG_PER_TILE,), jnp.float32),
            pltpu.VMEM_SHARED((ACC_ROWS,), jnp.float32),
            pltpu.SemaphoreType.DMA,
        ],
    )
    def k(dst_hbm, deg_hbm, dstv, ones_v, zrow, acc, sem):
        c = lax.axis_index("c")
        s = lax.axis_index("s")

        @pl.when(c == 0)
        def _():
            for i in range(DEG_PER_TILE // 16):
                zrow[pl.ds(i * 16, 16)] = jnp.zeros((16,), jnp.float32)
            for i in range(WIN // 16):
                ones_v[pl.ds(i * 16, 16)] = jnp.ones((16,), jnp.float32)
            pltpu.sync_copy(dst_hbm.at[pl.ds(s * NW, NW)], dstv)
            pltpu.sync_copy(zrow, acc.at[pl.ds(s * DEG_PER_TILE,
                                               DEG_PER_TILE)])
            plsc.subcore_barrier()

            def fire(w, _):
                pltpu.async_copy(ones_v, acc.at[dstv.at[w]], sem, add=True)
                return 0

            lax.fori_loop(0, NW, fire, 0)

            def drain(w, _):
                pltpu.make_async_copy(ones_v, acc.at[dstv.at[w]], sem).wait()
                return 0

            lax.fori_loop(0, NW, drain, 0)
            plsc.subcore_barrier()
            pltpu.sync_copy(acc.at[pl.ds(s * DEG_PER_TILE, DEG_PER_TILE)],
                            deg_hbm.at[pl.ds(s * DEG_PER_TILE,
                                             DEG_PER_TILE)])

    return k(dst2d)


def _msgpass_call(src2d, dst2d, ulo, uhi):
    @functools.partial(
        pl.kernel,
        out_type=(jax.ShapeDtypeStruct((N, HH), jnp.float32),
                  jax.ShapeDtypeStruct((N, HH), jnp.float32)),
        mesh=_sc_mesh(),
        scratch_types=[
            pltpu.VMEM((2, CH, WIN), jnp.int32),
            pltpu.VMEM((2, CH, WIN), jnp.int32),
            pltpu.VMEM((2, WIN, HH), jnp.float32),
            pltpu.VMEM_SHARED((ACC_ROWS, HH), jnp.float32),
            pltpu.SemaphoreType.DMA,
            pltpu.SemaphoreType.DMA,
            pltpu.SemaphoreType.DMA,
            pltpu.SemaphoreType.DMA,
            pltpu.SemaphoreType.DMA,
            pltpu.SemaphoreType.DMA,
        ],
    )
    def k(src_hbm, dst_hbm, ulo_hbm, uhi_hbm, olo_hbm, ohi_hbm,
          srcv, dstv, buf, acc, sem0, sem1, semi0, semi1, ssem0, ssem1):
        c = lax.axis_index("c")
        s = lax.axis_index("s")
        sems = (sem0, sem1)
        semis = (semi0, semi1)
        ssems = (ssem0, ssem1)

        def load_idx(k_chunk, slot):
            base = s * NW + k_chunk * CH
            pltpu.async_copy(src_hbm.at[pl.ds(base, CH)], srcv.at[slot],
                             semis[slot])
            pltpu.async_copy(dst_hbm.at[pl.ds(base, CH)], dstv.at[slot],
                             semis[slot])

        def wait_idx(k_chunk, slot):
            base = s * NW + k_chunk * CH
            pltpu.make_async_copy(src_hbm.at[pl.ds(base, CH)], srcv.at[slot],
                                  semis[slot]).wait()
            pltpu.make_async_copy(dst_hbm.at[pl.ds(base, CH)], dstv.at[slot],
                                  semis[slot]).wait()

        def half(u_hbm, o_hbm):
            pltpu.sync_copy(u_hbm.at[pl.ds(s * ROWS_PER_TILE, ROWS_PER_TILE)],
                            acc.at[pl.ds(s * ROWS_PER_TILE, ROWS_PER_TILE)])

            @pl.when(s == 0)
            def _():
                pltpu.sync_copy(u_hbm.at[pl.ds(TAIL_OFF, TAIL)],
                                acc.at[pl.ds(TAIL_OFF, TAIL)])
            load_idx(0, 0)
            load_idx(1, 1)
            plsc.subcore_barrier()

            def drain_tail():
                for g in range(2):
                    pltpu.make_async_copy(buf.at[g], acc.at[dstv.at[0, 0]],
                                          ssems[g]).wait()

            def chunk_pair(i, _):
                for b in range(2):
                    kc = i * 2 + b
                    wait_idx(kc, b)
                    if b == 1:
                        drain_tail()
                    else:
                        pl.when(i > 0)(drain_tail)
                    pltpu.async_copy(u_hbm.at[srcv.at[b, 0]], buf.at[0], sem0)

                    def wbody(jj, _, b=b):
                        for g in range(2):
                            j = jj * 2 + g
                            gn = 1 - g
                            pltpu.make_async_copy(u_hbm.at[srcv.at[b, j]],
                                                  buf.at[g], sems[g]).wait()
                            pltpu.async_copy(buf.at[g], acc.at[dstv.at[b, j]],
                                             ssems[g], add=True)

                            @pl.when(j + 1 < CH)
                            def _(g=g, gn=gn, j=j, b=b):
                                @pl.when(j >= 1)
                                def _():
                                    pltpu.make_async_copy(
                                        buf.at[gn], acc.at[dstv.at[b, 0]],
                                        ssems[gn]).wait()
                                pltpu.async_copy(u_hbm.at[srcv.at[b, j + 1]],
                                                 buf.at[gn], sems[gn])
                        return 0

                    lax.fori_loop(0, CH // 2, wbody, 0)

                    @pl.when(kc + 2 < NCHUNK)
                    def _():
                        load_idx(kc + 2, b)
                return 0

            lax.fori_loop(0, NCHUNK // 2, chunk_pair, 0)
            drain_tail()
            plsc.subcore_barrier()
            pltpu.sync_copy(acc.at[pl.ds(s * ROWS_PER_TILE, ROWS_PER_TILE)],
                            o_hbm.at[pl.ds(s * ROWS_PER_TILE, ROWS_PER_TILE)])

            @pl.when(s == 0)
            def _():
                pltpu.sync_copy(acc.at[pl.ds(TAIL_OFF, TAIL)],
                                o_hbm.at[pl.ds(TAIL_OFF, TAIL)])

        pl.when(c == 0)(lambda: half(ulo_hbm, olo_hbm))
        pl.when(c == 1)(lambda: half(uhi_hbm, ohi_hbm))

    return k(src2d, dst2d, ulo, uhi)


def _mm_pre_call(x, W0, deg2):
    def body(x_ref, w_ref, deg_ref, olo_ref, ohi_ref):
        t = jnp.dot(x_ref[...], w_ref[...], preferred_element_type=jnp.float32)
        dinv = lax.rsqrt(deg_ref[...] + 1.0)
        u = t * dinv
        olo_ref[...] = u[:, :HH]
        ohi_ref[...] = u[:, HH:]

    return pl.pallas_call(
        body,
        grid=(NBLK,),
        in_specs=[
            pl.BlockSpec((BLK, D), lambda i: (i, 0)),
            pl.BlockSpec((D, H), lambda i: (0, 0)),
            pl.BlockSpec((BLK, 1), lambda i: (i, 0)),
        ],
        out_specs=[
            pl.BlockSpec((BLK, HH), lambda i: (i, 0)),
            pl.BlockSpec((BLK, HH), lambda i: (i, 0)),
        ],
        out_shape=[
            jax.ShapeDtypeStruct((N, HH), jnp.float32),
            jax.ShapeDtypeStruct((N, HH), jnp.float32),
        ],
    )(x, W0, deg2)


def _post(slo, shi, dinv, b, g, be):
    s = jnp.concatenate([slo, shi], axis=1)
    hpre = s * dinv + b
    mu = jnp.mean(hpre, axis=-1, keepdims=True)
    var = jnp.mean((hpre - mu) ** 2, axis=-1, keepdims=True)
    h = (hpre - mu) / jnp.sqrt(var + 1e-5) * g + be
    return jnp.maximum(h, 0.0)


def _post_pre_call(slo, shi, deg2, b, g, be, W):
    def body(slo_ref, shi_ref, deg_ref, b_ref, g_ref, be_ref, w_ref,
             olo_ref, ohi_ref):
        dinv = lax.rsqrt(deg_ref[...] + 1.0)
        h = _post(slo_ref[...], shi_ref[...], dinv, b_ref[...], g_ref[...],
                  be_ref[...])
        t = jnp.dot(h, w_ref[...], preferred_element_type=jnp.float32)
        u = t * dinv
        olo_ref[...] = u[:, :HH]
        ohi_ref[...] = u[:, HH:]

    return pl.pallas_call(
        body,
        grid=(NBLK,),
        in_specs=[
            pl.BlockSpec((BLK, HH), lambda i: (i, 0)),
            pl.BlockSpec((BLK, HH), lambda i: (i, 0)),
            pl.BlockSpec((BLK, 1), lambda i: (i, 0)),
            pl.BlockSpec((1, H), lambda i: (0, 0)),
            pl.BlockSpec((1, H), lambda i: (0, 0)),
            pl.BlockSpec((1, H), lambda i: (0, 0)),
            pl.BlockSpec((H, H), lambda i: (0, 0)),
        ],
        out_specs=[
            pl.BlockSpec((BLK, HH), lambda i: (i, 0)),
            pl.BlockSpec((BLK, HH), lambda i: (i, 0)),
        ],
        out_shape=[
            jax.ShapeDtypeStruct((N, HH), jnp.float32),
            jax.ShapeDtypeStruct((N, HH), jnp.float32),
        ],
    )(slo, shi, deg2, b, g, be, W)


def _final_call(slo, shi, deg2, b, g, be, batch_row, batch_col,
                cW1, cb1, cW2, cb2, cW3, cb3):
    def body(slo_ref, shi_ref, deg_ref, b_ref, g_ref, be_ref,
             brow_ref, bcol_ref, cw1_ref, cb1_ref, cw2_ref, cb2_ref,
             cw3_ref, cb3_ref, out_ref, mean_acc, cnt_acc, max_acc):
        i = pl.program_id(0)

        @pl.when(i == 0)
        def _():
            mean_acc[...] = jnp.zeros((G, H), jnp.float32)
            cnt_acc[...] = jnp.zeros((G, 1), jnp.float32)
            max_acc[...] = jnp.zeros((G, H), jnp.float32)

        dinv = lax.rsqrt(deg_ref[...] + 1.0)
        h = _post(slo_ref[...], shi_ref[...], dinv, b_ref[...], g_ref[...],
                  be_ref[...])
        brow = brow_ref[...].reshape(1, BLK)
        iota_g = lax.broadcasted_iota(jnp.int32, (G, 1), 0)
        mask = (brow == iota_g).astype(jnp.float32)
        mean_acc[...] += jnp.dot(mask, h, preferred_element_type=jnp.float32)
        cnt_acc[...] += jnp.sum(mask, axis=1, keepdims=True)

        bcol = bcol_ref[...]
        g_lo = bcol_ref[0, 0]
        g_hi = bcol_ref[BLK - 1, 0]

        def gbody(gg, _):
            hm = jnp.where(bcol == gg, h, 0.0)
            bm = jnp.max(hm, axis=0, keepdims=True)
            max_acc[pl.ds(gg, 1), :] = jnp.maximum(max_acc[pl.ds(gg, 1), :],
                                                   bm)
            return 0

        lax.fori_loop(g_lo, g_hi + 1, gbody, 0)

        @pl.when(i == NBLK - 1)
        def _():
            cnt = jnp.maximum(cnt_acc[...], 1.0)
            z = jnp.concatenate([mean_acc[...] / cnt, max_acc[...]], axis=1)
            z1 = jnp.maximum(
                jnp.dot(z, cw1_ref[...], preferred_element_type=jnp.float32)
                + cb1_ref[...], 0.0)
            z2 = jnp.maximum(
                jnp.dot(z1, cw2_ref[...], preferred_element_type=jnp.float32)
                + cb2_ref[...], 0.0)
            out_ref[...] = (jnp.dot(z2, cw3_ref[...],
                                    preferred_element_type=jnp.float32)
                            + cb3_ref[...])

    return pl.pallas_call(
        body,
        grid=(NBLK,),
        in_specs=[
            pl.BlockSpec((BLK, HH), lambda i: (i, 0)),
            pl.BlockSpec((BLK, HH), lambda i: (i, 0)),
            pl.BlockSpec((BLK, 1), lambda i: (i, 0)),
            pl.BlockSpec((1, H), lambda i: (0, 0)),
            pl.BlockSpec((1, H), lambda i: (0, 0)),
            pl.BlockSpec((1, H), lambda i: (0, 0)),
            pl.BlockSpec((1, 1, BLK), lambda i: (i, 0, 0)),
            pl.BlockSpec((BLK, 1), lambda i: (i, 0)),
            pl.BlockSpec((2 * H, H), lambda i: (0, 0)),
            pl.BlockSpec((1, H), lambda i: (0, 0)),
            pl.BlockSpec((H, HH), lambda i: (0, 0)),
            pl.BlockSpec((1, HH), lambda i: (0, 0)),
            pl.BlockSpec((HH, 1), lambda i: (0, 0)),
            pl.BlockSpec((1, 1), lambda i: (0, 0)),
        ],
        out_specs=pl.BlockSpec((G, 1), lambda i: (0, 0)),
        out_shape=jax.ShapeDtypeStruct((G, 1), jnp.float32),
        scratch_shapes=[
            pltpu.VMEM((G, H), jnp.float32),
            pltpu.VMEM((G, 1), jnp.float32),
            pltpu.VMEM((G, H), jnp.float32),
        ],
    )(slo, shi, deg2, b, g, be, batch_row, batch_col,
      cW1, cb1, cW2, cb2, cW3, cb3)


def kernel(x, edge_index, batch, W0, b0, W1, b1, W2, b2, g0, be0, g1, be1,
           g2, be2, cW1, cb1, cW2, cb2, cW3, cb3):
    pad_i = jnp.arange(PAD, dtype=jnp.int32)
    src_p = jnp.concatenate([edge_index[0], pad_i % N])
    dst_p = jnp.concatenate([edge_index[1], N + pad_i % NDUM])
    src2d = src_p.reshape(EP // WIN, WIN)
    dst2d = dst_p.reshape(EP // WIN, WIN)

    deg_raw = _deg_call(dst2d)
    deg2 = deg_raw[:N].reshape(N, 1)

    b0r, g0r, be0r = b0.reshape(1, H), g0.reshape(1, H), be0.reshape(1, H)
    b1r, g1r, be1r = b1.reshape(1, H), g1.reshape(1, H), be1.reshape(1, H)
    b2r, g2r, be2r = b2.reshape(1, H), g2.reshape(1, H), be2.reshape(1, H)

    ulo, uhi = _mm_pre_call(x, W0, deg2)
    slo, shi = _msgpass_call(src2d, dst2d, ulo, uhi)
    ulo, uhi = _post_pre_call(slo, shi, deg2, b0r, g0r, be0r, W1)
    slo, shi = _msgpass_call(src2d, dst2d, ulo, uhi)
    ulo, uhi = _post_pre_call(slo, shi, deg2, b1r, g1r, be1r, W2)
    slo, shi = _msgpass_call(src2d, dst2d, ulo, uhi)

    out = _final_call(slo, shi, deg2, b2r, g2r, be2r,
                      batch.reshape(NBLK, 1, BLK), batch.reshape(N, 1),
                      cW1, cb1.reshape(1, H), cW2, cb2.reshape(1, HH),
                      cW3, cb3.reshape(1, 1))
    return out.reshape(G)


# speedup vs baseline: 1.1431x; 1.1431x over previous
import functools

import jax
import jax.numpy as jnp
from jax import lax
from jax.experimental import pallas as pl
from jax.experimental.pallas import tpu as pltpu
from jax.experimental.pallas import tpu_sc as plsc

N = 10000
E = 320000
D = 128
H = 256
HH = 128
G = 64

NTILES = 16
WIN = 128
NW = 160
CH = 16
NCHUNK = NW // CH
EP = NTILES * NW * WIN
PAD = EP - E
NDUM = 240
ACC_ROWS = N + NDUM
ROWS_PER_TILE = 624
TAIL = N - NTILES * ROWS_PER_TILE
TAIL_OFF = NTILES * ROWS_PER_TILE
DEG_PER_TILE = ACC_ROWS // NTILES

BLK = 1000
NBLK = N // BLK


def _sc_mesh():
    return plsc.VectorSubcoreMesh(
        core_axis_name="c", subcore_axis_name="s", num_cores=2,
        num_subcores=NTILES)


def _deg_call(dst2d):
    @functools.partial(
        pl.kernel,
        out_type=jax.ShapeDtypeStruct((ACC_ROWS,), jnp.float32),
        mesh=_sc_mesh(),
        scratch_types=[
            pltpu.VMEM((NW, WIN), jnp.int32),
            pltpu.VMEM((WIN,), jnp.float32),
            pltpu.VMEM((DEG_PER_TILE,), jnp.float32),
            pltpu.VMEM_SHARED((ACC_ROWS,), jnp.float32),
            pltpu.SemaphoreType.DMA,
        ],
    )
    def k(dst_hbm, deg_hbm, dstv, ones_v, zrow, acc, sem):
        c = lax.axis_index("c")
        s = lax.axis_index("s")

        @pl.when(c == 0)
        def _():
            for i in range(DEG_PER_TILE // 16):
                zrow[pl.ds(i * 16, 16)] = jnp.zeros((16,), jnp.float32)
            for i in range(WIN // 16):
                ones_v[pl.ds(i * 16, 16)] = jnp.ones((16,), jnp.float32)
            pltpu.sync_copy(dst_hbm.at[pl.ds(s * NW, NW)], dstv)
            pltpu.sync_copy(zrow, acc.at[pl.ds(s * DEG_PER_TILE,
                                               DEG_PER_TILE)])
            plsc.subcore_barrier()

            def fire(w, _):
                pltpu.async_copy(ones_v, acc.at[dstv.at[w]], sem, add=True)
                return 0

            lax.fori_loop(0, NW, fire, 0)

            def drain(w, _):
                pltpu.make_async_copy(ones_v, acc.at[dstv.at[w]], sem).wait()
                return 0

            lax.fori_loop(0, NW, drain, 0)
            plsc.subcore_barrier()
            pltpu.sync_copy(acc.at[pl.ds(s * DEG_PER_TILE, DEG_PER_TILE)],
                            deg_hbm.at[pl.ds(s * DEG_PER_TILE,
                                             DEG_PER_TILE)])

    return k(dst2d)


def _msgpass_call(src2d, dst2d, ulo, uhi):
    @functools.partial(
        pl.kernel,
        out_type=(jax.ShapeDtypeStruct((N, HH), jnp.float32),
                  jax.ShapeDtypeStruct((N, HH), jnp.float32)),
        mesh=_sc_mesh(),
        scratch_types=[
            pltpu.VMEM((2, CH, WIN), jnp.int32),
            pltpu.VMEM((2, CH, WIN), jnp.int32),
            pltpu.VMEM((2, WIN, HH), jnp.float32),
            pltpu.VMEM_SHARED((ACC_ROWS, HH), jnp.float32),
            pltpu.SemaphoreType.DMA,
            pltpu.SemaphoreType.DMA,
            pltpu.SemaphoreType.DMA,
            pltpu.SemaphoreType.DMA,
            pltpu.SemaphoreType.DMA,
            pltpu.SemaphoreType.DMA,
        ],
    )
    def k(src_hbm, dst_hbm, ulo_hbm, uhi_hbm, olo_hbm, ohi_hbm,
          srcv, dstv, buf, acc, sem0, sem1, semi0, semi1, ssem0, ssem1):
        c = lax.axis_index("c")
        s = lax.axis_index("s")
        sems = (sem0, sem1)
        semis = (semi0, semi1)
        ssems = (ssem0, ssem1)

        def load_idx(k_chunk, slot):
            base = s * NW + k_chunk * CH
            pltpu.async_copy(src_hbm.at[pl.ds(base, CH)], srcv.at[slot],
                             semis[slot])
            pltpu.async_copy(dst_hbm.at[pl.ds(base, CH)], dstv.at[slot],
                             semis[slot])

        def wait_idx(k_chunk, slot):
            base = s * NW + k_chunk * CH
            pltpu.make_async_copy(src_hbm.at[pl.ds(base, CH)], srcv.at[slot],
                                  semis[slot]).wait()
            pltpu.make_async_copy(dst_hbm.at[pl.ds(base, CH)], dstv.at[slot],
                                  semis[slot]).wait()

        def half(u_hbm, o_hbm):
            pltpu.sync_copy(u_hbm.at[pl.ds(s * ROWS_PER_TILE, ROWS_PER_TILE)],
                            acc.at[pl.ds(s * ROWS_PER_TILE, ROWS_PER_TILE)])

            @pl.when(s == 0)
            def _():
                pltpu.sync_copy(u_hbm.at[pl.ds(TAIL_OFF, TAIL)],
                                acc.at[pl.ds(TAIL_OFF, TAIL)])
            load_idx(0, 0)
            load_idx(1, 1)
            plsc.subcore_barrier()

            def chunk_pair(i, _):
                for b in range(2):
                    kc = i * 2 + b
                    wait_idx(kc, b)
                    pltpu.async_copy(u_hbm.at[srcv.at[b, 0]], buf.at[0], sem0)
                    pltpu.async_copy(u_hbm.at[srcv.at[b, 1]], buf.at[1], sem1)

                    def wbody(jj, _, b=b):
                        for g in range(2):
                            j = jj * 2 + g
                            pltpu.make_async_copy(u_hbm.at[srcv.at[b, j]],
                                                  buf.at[g], sems[g]).wait()
                            pltpu.sync_copy(buf.at[g], acc.at[dstv.at[b, j]],
                                            add=True)

                            @pl.when(j + 2 < CH)
                            def _(g=g, j=j, b=b):
                                pltpu.async_copy(u_hbm.at[srcv.at[b, j + 2]],
                                                 buf.at[g], sems[g])
                        return 0

                    lax.fori_loop(0, CH // 2, wbody, 0)

                    @pl.when(kc + 2 < NCHUNK)
                    def _():
                        load_idx(kc + 2, b)
                return 0

            lax.fori_loop(0, NCHUNK // 2, chunk_pair, 0)
            plsc.subcore_barrier()
            pltpu.sync_copy(acc.at[pl.ds(s * ROWS_PER_TILE, ROWS_PER_TILE)],
                            o_hbm.at[pl.ds(s * ROWS_PER_TILE, ROWS_PER_TILE)])

            @pl.when(s == 0)
            def _():
                pltpu.sync_copy(acc.at[pl.ds(TAIL_OFF, TAIL)],
                                o_hbm.at[pl.ds(TAIL_OFF, TAIL)])

        pl.when(c == 0)(lambda: half(ulo_hbm, olo_hbm))
        pl.when(c == 1)(lambda: half(uhi_hbm, ohi_hbm))

    return k(src2d, dst2d, ulo, uhi)


def _mm_pre_call(x, W0, deg2):
    def body(x_ref, w_ref, deg_ref, olo_ref, ohi_ref):
        t = jnp.dot(x_ref[...], w_ref[...], preferred_element_type=jnp.float32)
        dinv = lax.rsqrt(deg_ref[...] + 1.0)
        u = t * dinv
        olo_ref[...] = u[:, :HH]
        ohi_ref[...] = u[:, HH:]

    return pl.pallas_call(
        body,
        grid=(NBLK,),
        in_specs=[
            pl.BlockSpec((BLK, D), lambda i: (i, 0)),
            pl.BlockSpec((D, H), lambda i: (0, 0)),
            pl.BlockSpec((BLK, 1), lambda i: (i, 0)),
        ],
        out_specs=[
            pl.BlockSpec((BLK, HH), lambda i: (i, 0)),
            pl.BlockSpec((BLK, HH), lambda i: (i, 0)),
        ],
        out_shape=[
            jax.ShapeDtypeStruct((N, HH), jnp.float32),
            jax.ShapeDtypeStruct((N, HH), jnp.float32),
        ],
    )(x, W0, deg2)


def _post(slo, shi, dinv, b, g, be):
    s = jnp.concatenate([slo, shi], axis=1)
    hpre = s * dinv + b
    mu = jnp.mean(hpre, axis=-1, keepdims=True)
    var = jnp.mean((hpre - mu) ** 2, axis=-1, keepdims=True)
    h = (hpre - mu) / jnp.sqrt(var + 1e-5) * g + be
    return jnp.maximum(h, 0.0)


def _post_pre_call(slo, shi, deg2, b, g, be, W):
    def body(slo_ref, shi_ref, deg_ref, b_ref, g_ref, be_ref, w_ref,
             olo_ref, ohi_ref):
        dinv = lax.rsqrt(deg_ref[...] + 1.0)
        h = _post(slo_ref[...], shi_ref[...], dinv, b_ref[...], g_ref[...],
                  be_ref[...])
        t = jnp.dot(h, w_ref[...], preferred_element_type=jnp.float32)
        u = t * dinv
        olo_ref[...] = u[:, :HH]
        ohi_ref[...] = u[:, HH:]

    return pl.pallas_call(
        body,
        grid=(NBLK,),
        in_specs=[
            pl.BlockSpec((BLK, HH), lambda i: (i, 0)),
            pl.BlockSpec((BLK, HH), lambda i: (i, 0)),
            pl.BlockSpec((BLK, 1), lambda i: (i, 0)),
            pl.BlockSpec((1, H), lambda i: (0, 0)),
            pl.BlockSpec((1, H), lambda i: (0, 0)),
            pl.BlockSpec((1, H), lambda i: (0, 0)),
            pl.BlockSpec((H, H), lambda i: (0, 0)),
        ],
        out_specs=[
            pl.BlockSpec((BLK, HH), lambda i: (i, 0)),
            pl.BlockSpec((BLK, HH), lambda i: (i, 0)),
        ],
        out_shape=[
            jax.ShapeDtypeStruct((N, HH), jnp.float32),
            jax.ShapeDtypeStruct((N, HH), jnp.float32),
        ],
    )(slo, shi, deg2, b, g, be, W)


def _final_call(slo, shi, deg2, b, g, be, batch_row, batch_col,
                cW1, cb1, cW2, cb2, cW3, cb3):
    def body(slo_ref, shi_ref, deg_ref, b_ref, g_ref, be_ref,
             brow_ref, bcol_ref, cw1_ref, cb1_ref, cw2_ref, cb2_ref,
             cw3_ref, cb3_ref, out_ref, mean_acc, cnt_acc, max_acc):
        i = pl.program_id(0)

        @pl.when(i == 0)
        def _():
            mean_acc[...] = jnp.zeros((G, H), jnp.float32)
            cnt_acc[...] = jnp.zeros((G, 1), jnp.float32)
            max_acc[...] = jnp.zeros((G, H), jnp.float32)

        dinv = lax.rsqrt(deg_ref[...] + 1.0)
        h = _post(slo_ref[...], shi_ref[...], dinv, b_ref[...], g_ref[...],
                  be_ref[...])
        brow = brow_ref[...].reshape(1, BLK)
        iota_g = lax.broadcasted_iota(jnp.int32, (G, 1), 0)
        mask = (brow == iota_g).astype(jnp.float32)
        mean_acc[...] += jnp.dot(mask, h, preferred_element_type=jnp.float32)
        cnt_acc[...] += jnp.sum(mask, axis=1, keepdims=True)

        bcol = bcol_ref[...]
        g_lo = bcol_ref[0, 0]
        g_hi = bcol_ref[BLK - 1, 0]

        def gbody(gg, _):
            hm = jnp.where(bcol == gg, h, 0.0)
            bm = jnp.max(hm, axis=0, keepdims=True)
            max_acc[pl.ds(gg, 1), :] = jnp.maximum(max_acc[pl.ds(gg, 1), :],
                                                   bm)
            return 0

        lax.fori_loop(g_lo, g_hi + 1, gbody, 0)

        @pl.when(i == NBLK - 1)
        def _():
            cnt = jnp.maximum(cnt_acc[...], 1.0)
            z = jnp.concatenate([mean_acc[...] / cnt, max_acc[...]], axis=1)
            z1 = jnp.maximum(
                jnp.dot(z, cw1_ref[...], preferred_element_type=jnp.float32)
                + cb1_ref[...], 0.0)
            z2 = jnp.maximum(
                jnp.dot(z1, cw2_ref[...], preferred_element_type=jnp.float32)
                + cb2_ref[...], 0.0)
            out_ref[...] = (jnp.dot(z2, cw3_ref[...],
                                    preferred_element_type=jnp.float32)
                            + cb3_ref[...])

    return pl.pallas_call(
        body,
        grid=(NBLK,),
        in_specs=[
            pl.BlockSpec((BLK, HH), lambda i: (i, 0)),
            pl.BlockSpec((BLK, HH), lambda i: (i, 0)),
            pl.BlockSpec((BLK, 1), lambda i: (i, 0)),
            pl.BlockSpec((1, H), lambda i: (0, 0)),
            pl.BlockSpec((1, H), lambda i: (0, 0)),
            pl.BlockSpec((1, H), lambda i: (0, 0)),
            pl.BlockSpec((1, 1, BLK), lambda i: (i, 0, 0)),
            pl.BlockSpec((BLK, 1), lambda i: (i, 0)),
            pl.BlockSpec((2 * H, H), lambda i: (0, 0)),
            pl.BlockSpec((1, H), lambda i: (0, 0)),
            pl.BlockSpec((H, HH), lambda i: (0, 0)),
            pl.BlockSpec((1, HH), lambda i: (0, 0)),
            pl.BlockSpec((HH, 1), lambda i: (0, 0)),
            pl.BlockSpec((1, 1), lambda i: (0, 0)),
        ],
        out_specs=pl.BlockSpec((G, 1), lambda i: (0, 0)),
        out_shape=jax.ShapeDtypeStruct((G, 1), jnp.float32),
        scratch_shapes=[
            pltpu.VMEM((G, H), jnp.float32),
            pltpu.VMEM((G, 1), jnp.float32),
            pltpu.VMEM((G, H), jnp.float32),
        ],
    )(slo, shi, deg2, b, g, be, batch_row, batch_col,
      cW1, cb1, cW2, cb2, cW3, cb3)


def kernel(x, edge_index, batch, W0, b0, W1, b1, W2, b2, g0, be0, g1, be1,
           g2, be2, cW1, cb1, cW2, cb2, cW3, cb3):
    pad_i = jnp.arange(PAD, dtype=jnp.int32)
    src_p = jnp.concatenate([edge_index[0], pad_i % N])
    dst_p = jnp.concatenate([edge_index[1], N + pad_i % NDUM])
    src2d = src_p.reshape(EP // WIN, WIN)
    dst2d = dst_p.reshape(EP // WIN, WIN)

    deg_raw = _deg_call(dst2d)
    deg2 = deg_raw[:N].reshape(N, 1)

    b0r, g0r, be0r = b0.reshape(1, H), g0.reshape(1, H), be0.reshape(1, H)
    b1r, g1r, be1r = b1.reshape(1, H), g1.reshape(1, H), be1.reshape(1, H)
    b2r, g2r, be2r = b2.reshape(1, H), g2.reshape(1, H), be2.reshape(1, H)

    ulo, uhi = _mm_pre_call(x, W0, deg2)
    slo, shi = _msgpass_call(src2d, dst2d, ulo, uhi)
    ulo, uhi = _post_pre_call(slo, shi, deg2, b0r, g0r, be0r, W1)
    slo, shi = _msgpass_call(src2d, dst2d, ulo, uhi)
    ulo, uhi = _post_pre_call(slo, shi, deg2, b1r, g1r, be1r, W2)
    slo, shi = _msgpass_call(src2d, dst2d, ulo, uhi)

    out = _final_call(slo, shi, deg2, b2r, g2r, be2r,
                      batch.reshape(NBLK, 1, BLK), batch.reshape(N, 1),
                      cW1, cb1.reshape(1, H), cW2, cb2.reshape(1, HH),
                      cW3, cb3.reshape(1, 1))
    return out.reshape(G)
